# Initial kernel scaffold; baseline (speedup 1.0000x reference)
import jax
import jax.numpy as jnp
from jax.experimental import pallas as pl


def kernel(x, edge_index, W_self1, W_neigh1, b1, W_self2, W_neigh2, b2):
    raise NotImplementedError("write your pallas kernel here")


# speedup vs baseline: 11.5332x; 11.5332x over previous
import functools

import jax
import jax.numpy as jnp
from jax import lax
from jax.experimental import pallas as pl
from jax.experimental.pallas import tpu as pltpu
from jax.experimental.pallas import tpu_sc as plsc

N = 10000
E = 320000
D = 128
DH = D // 2

NC = 2
NS = 16
NP = 10240
C = 80
EPT = E // NS
CPT = EPT // C
NBUF = 5
ROWS_T = NP // NS

_mesh = plsc.VectorSubcoreMesh(core_axis_name="c", subcore_axis_name="s")
_sc_params = pltpu.CompilerParams(use_tc_tiling_on_sc=False)


def _agg_body(with_deg, ys_hbm, src_hbm, dst_hbm, zf_hbm, *rest):
    if with_deg:
        (zd_hbm, aggp_hbm, degp_hbm, src_v, dst_v,
         b0, b1, b2, b3, b4, agg_s, ones_v, deg_s,
         s0, s1, s2, s3, s4) = rest
    else:
        (aggp_hbm, src_v, dst_v,
         b0, b1, b2, b3, b4, agg_s,
         s0, s1, s2, s3, s4) = rest
    bufs = (b0, b1, b2, b3, b4)
    sems = (s0, s1, s2, s3, s4)

    cid = lax.axis_index("c")
    sid = lax.axis_index("s")

    rbase = sid * ROWS_T
    pltpu.sync_copy(zf_hbm.at[sid], agg_s.at[pl.ds(rbase, ROWS_T)])
    if with_deg:
        @pl.when(cid == 0)
        def _():
            pltpu.sync_copy(zd_hbm.at[pl.ds(rbase, ROWS_T)],
                            deg_s.at[pl.ds(rbase, ROWS_T)])
        for i in range(C // 16):
            ones_v[pl.ds(i * 16, 16)] = jnp.ones((16,), jnp.float32)

    pltpu.sync_copy(src_hbm.at[sid], src_v)
    pltpu.sync_copy(dst_hbm.at[sid], dst_v)

    off = cid * NP

    def rebase(c, carry):
        for g in range(C // 16):
            sl = pl.ds(g * 16, 16)
            src_v[c, sl] = src_v[c, sl] + off
        return carry

    lax.fori_loop(0, CPT, rebase, 0)

    plsc.subcore_barrier()

    for b in range(NBUF):
        pltpu.async_copy(ys_hbm.at[src_v.at[b]], bufs[b], sems[b])

    def step(c, b):
        pltpu.make_async_copy(ys_hbm.at[src_v.at[c]], bufs[b], sems[b]).wait()
        pltpu.sync_copy(bufs[b], agg_s.at[dst_v.at[c]], add=True)
        if with_deg:
            @pl.when(cid == 0)
            def _():
                pltpu.sync_copy(ones_v, deg_s.at[dst_v.at[c]], add=True)

    def outer(i, carry):
        c0 = i * NBUF
        for b in range(NBUF):
            step(c0 + b, b)
            pltpu.async_copy(ys_hbm.at[src_v.at[c0 + b + NBUF]], bufs[b], sems[b])
        return carry

    lax.fori_loop(0, CPT // NBUF - 1, outer, 0)
    for b in range(NBUF):
        step(CPT - NBUF + b, b)

    plsc.subcore_barrier()

    pltpu.sync_copy(agg_s.at[pl.ds(rbase, ROWS_T)],
                    aggp_hbm.at[cid, pl.ds(rbase, ROWS_T)])
    if with_deg:
        @pl.when(cid == 0)
        def _():
            pltpu.sync_copy(deg_s.at[pl.ds(rbase, ROWS_T)],
                            degp_hbm.at[pl.ds(rbase, ROWS_T)])


def _make_agg(with_deg):
    out_type = [jax.ShapeDtypeStruct((NC, NP, DH), jnp.float32)]
    if with_deg:
        out_type.append(jax.ShapeDtypeStruct((NP,), jnp.float32))
    scratch = [
        pltpu.VMEM((CPT, C), jnp.int32),
        pltpu.VMEM((CPT, C), jnp.int32),
    ]
    scratch += [pltpu.VMEM((C, DH), jnp.float32) for _ in range(NBUF)]
    scratch += [pltpu.VMEM_SHARED((NP, DH), jnp.float32)]
    if with_deg:
        scratch += [pltpu.VMEM((C,), jnp.float32),
                    pltpu.VMEM_SHARED((NP,), jnp.float32)]
    scratch += [pltpu.SemaphoreType.DMA for _ in range(NBUF)]
    return pl.kernel(
        functools.partial(_agg_body, with_deg),
        out_type=out_type,
        mesh=_mesh,
        scratch_types=scratch,
        compiler_params=_sc_params,
    )


_agg_with_deg = _make_agg(True)
_agg_no_deg = _make_agg(False)


def _l1_body(x_ref, ws_ref, wn_ref, b_ref, s_ref, ys_ref):
    xv = x_ref[...]
    s_ref[...] = jnp.dot(xv, ws_ref[...], preferred_element_type=jnp.float32) + b_ref[...]
    y = jnp.dot(xv, wn_ref[...], preferred_element_type=jnp.float32)
    ys_ref[0:NP, :] = y[:, 0:DH]
    ys_ref[NP:2 * NP, :] = y[:, DH:D]


def _l2_body(s1_ref, aggp_ref, deg_ref, ws_ref, wn_ref, b_ref,
             s2_ref, ys2_ref, rdeg_ref):
    rdeg = 1.0 / jnp.maximum(deg_ref[...], 1.0)
    agg = jnp.concatenate([aggp_ref[0], aggp_ref[1]], axis=1)
    h1 = jnp.maximum(s1_ref[...] + agg * rdeg, 0.0)
    s2_ref[...] = jnp.dot(h1, ws_ref[...], preferred_element_type=jnp.float32) + b_ref[...]
    y2 = jnp.dot(h1, wn_ref[...], preferred_element_type=jnp.float32)
    ys2_ref[0:NP, :] = y2[:, 0:DH]
    ys2_ref[NP:2 * NP, :] = y2[:, DH:D]
    rdeg_ref[...] = rdeg


def _fin_body(s2_ref, aggp_ref, rdeg_ref, out_ref):
    agg = jnp.concatenate([aggp_ref[0], aggp_ref[1]], axis=1)
    out_ref[...] = s2_ref[...] + agg * rdeg_ref[...]


def kernel(x, edge_index, W_self1, W_neigh1, b1, W_self2, W_neigh2, b2):
    f32 = jnp.float32
    src3d = edge_index[0].reshape(NS, CPT, C)
    dst3d = edge_index[1].reshape(NS, CPT, C)
    xp = jnp.pad(x, ((0, NP - N), (0, 0)))
    zf = jnp.zeros((NS, ROWS_T, DH), f32)
    zd = jnp.zeros((NP,), f32)
    b1r = b1.reshape(1, D)
    b2r = b2.reshape(1, D)

    s1, ys1 = pl.pallas_call(
        _l1_body,
        out_shape=[jax.ShapeDtypeStruct((NP, D), f32),
                   jax.ShapeDtypeStruct((2 * NP, DH), f32)],
    )(xp, W_self1, W_neigh1, b1r)

    aggp1, degp = _agg_with_deg(ys1, src3d, dst3d, zf, zd)
    deg2d = degp.reshape(NP, 1)

    s2, ys2, rdeg = pl.pallas_call(
        _l2_body,
        out_shape=[jax.ShapeDtypeStruct((NP, D), f32),
                   jax.ShapeDtypeStruct((2 * NP, DH), f32),
                   jax.ShapeDtypeStruct((NP, 1), f32)],
    )(s1, aggp1, deg2d, W_self2, W_neigh2, b2r)

    (aggp2,) = _agg_no_deg(ys2, src3d, dst3d, zf)

    out = pl.pallas_call(
        _fin_body,
        out_shape=jax.ShapeDtypeStruct((NP, D), f32),
    )(s2, aggp2, rdeg)
    return out[:N]


# speedup vs baseline: 11.5547x; 1.0019x over previous
import functools

import jax
import jax.numpy as jnp
from jax import lax
from jax.experimental import pallas as pl
from jax.experimental.pallas import tpu as pltpu
from jax.experimental.pallas import tpu_sc as plsc

N = 10000
E = 320000
D = 128
DH = D // 2

NC = 2
NS = 16
NP = 10240
C = 125
EPT = E // NS
CPT = EPT // C
NBUF = 5
ROWS_T = NP // NS

_mesh = plsc.VectorSubcoreMesh(core_axis_name="c", subcore_axis_name="s")
_sc_params = pltpu.CompilerParams(use_tc_tiling_on_sc=False)


def _agg_body(with_deg, ys_hbm, src_hbm, dst_hbm, zf_hbm, *rest):
    if with_deg:
        (zd_hbm, aggp_hbm, degp_hbm, src_v, dst_v,
         b0, b1, b2, b3, b4, agg_s, ones_v, deg_s,
         s0, s1, s2, s3, s4) = rest
    else:
        (aggp_hbm, src_v, dst_v,
         b0, b1, b2, b3, b4, agg_s,
         s0, s1, s2, s3, s4) = rest
    bufs = (b0, b1, b2, b3, b4)
    sems = (s0, s1, s2, s3, s4)

    cid = lax.axis_index("c")
    sid = lax.axis_index("s")

    rbase = sid * ROWS_T
    pltpu.sync_copy(zf_hbm.at[sid], agg_s.at[pl.ds(rbase, ROWS_T)])
    if with_deg:
        pltpu.sync_copy(zd_hbm.at[pl.ds(rbase, ROWS_T)],
                        deg_s.at[pl.ds(rbase, ROWS_T)])
        for i in range(8):
            ones_v[pl.ds(i * 16, 16)] = jnp.ones((16,), jnp.float32)

    pltpu.sync_copy(src_hbm.at[sid], src_v)
    pltpu.sync_copy(dst_hbm.at[sid], dst_v)

    plsc.subcore_barrier()

    yv = ys_hbm.at[pl.ds(cid * NP, NP)]

    for b in range(NBUF):
        pltpu.async_copy(yv.at[src_v.at[b]], bufs[b], sems[b])

    def step(c, b):
        pltpu.make_async_copy(yv.at[src_v.at[c]], bufs[b], sems[b]).wait()
        pltpu.sync_copy(bufs[b], agg_s.at[dst_v.at[c]], add=True)
        if with_deg:
            @pl.when((cid == 0) == (c < CPT // 2))
            def _():
                pltpu.sync_copy(ones_v.at[pl.ds(0, C)],
                                deg_s.at[dst_v.at[c]], add=True)

    def outer(i, carry):
        c0 = i * NBUF
        for b in range(NBUF):
            step(c0 + b, b)
            pltpu.async_copy(yv.at[src_v.at[c0 + b + NBUF]], bufs[b], sems[b])
        return carry

    lax.fori_loop(0, CPT // NBUF - 1, outer, 0)
    for b in range(NBUF):
        step(CPT - NBUF + b, b)

    plsc.subcore_barrier()

    pltpu.sync_copy(agg_s.at[pl.ds(rbase, ROWS_T)],
                    aggp_hbm.at[cid, pl.ds(rbase, ROWS_T)])
    if with_deg:
        pltpu.sync_copy(deg_s.at[pl.ds(rbase, ROWS_T)],
                        degp_hbm.at[cid, pl.ds(rbase, ROWS_T)])


def _make_agg(with_deg):
    out_type = [jax.ShapeDtypeStruct((NC, NP, DH), jnp.float32)]
    if with_deg:
        out_type.append(jax.ShapeDtypeStruct((NC, NP), jnp.float32))
    scratch = [
        pltpu.VMEM((CPT, C), jnp.int32),
        pltpu.VMEM((CPT, C), jnp.int32),
    ]
    scratch += [pltpu.VMEM((C, DH), jnp.float32) for _ in range(NBUF)]
    scratch += [pltpu.VMEM_SHARED((NP, DH), jnp.float32)]
    if with_deg:
        scratch += [pltpu.VMEM((128,), jnp.float32),
                    pltpu.VMEM_SHARED((NP,), jnp.float32)]
    scratch += [pltpu.SemaphoreType.DMA for _ in range(NBUF)]
    return pl.kernel(
        functools.partial(_agg_body, with_deg),
        out_type=out_type,
        mesh=_mesh,
        scratch_types=scratch,
        compiler_params=_sc_params,
    )


_agg_with_deg = _make_agg(True)
_agg_no_deg = _make_agg(False)


def _l1_body(x_ref, ws_ref, wn_ref, b_ref, s_ref, ys_ref):
    xv = x_ref[...]
    s_ref[...] = jnp.dot(xv, ws_ref[...], preferred_element_type=jnp.float32) + b_ref[...]
    y = jnp.dot(xv, wn_ref[...], preferred_element_type=jnp.float32)
    ys_ref[0:NP, :] = y[:, 0:DH]
    ys_ref[NP:2 * NP, :] = y[:, DH:D]


def _l2_body(s1_ref, aggp_ref, degt_ref, ws_ref, wn_ref, b_ref,
             s2_ref, ys2_ref, rdeg_ref):
    deg = degt_ref[:, 0:1] + degt_ref[:, 1:2]
    rdeg = 1.0 / jnp.maximum(deg, 1.0)
    agg = jnp.concatenate([aggp_ref[0], aggp_ref[1]], axis=1)
    h1 = jnp.maximum(s1_ref[...] + agg * rdeg, 0.0)
    s2_ref[...] = jnp.dot(h1, ws_ref[...], preferred_element_type=jnp.float32) + b_ref[...]
    y2 = jnp.dot(h1, wn_ref[...], preferred_element_type=jnp.float32)
    ys2_ref[0:NP, :] = y2[:, 0:DH]
    ys2_ref[NP:2 * NP, :] = y2[:, DH:D]
    rdeg_ref[...] = rdeg


def _fin_body(s2_ref, aggp_ref, rdeg_ref, out_ref):
    agg = jnp.concatenate([aggp_ref[0], aggp_ref[1]], axis=1)
    out_ref[...] = s2_ref[...] + agg * rdeg_ref[...]


def kernel(x, edge_index, W_self1, W_neigh1, b1, W_self2, W_neigh2, b2):
    f32 = jnp.float32
    src3d = edge_index[0].reshape(NS, CPT, C)
    dst3d = edge_index[1].reshape(NS, CPT, C)
    xp = jnp.pad(x, ((0, NP - N), (0, 0)))
    zf = jnp.zeros((NS, ROWS_T, DH), f32)
    zd = jnp.zeros((NP,), f32)
    b1r = b1.reshape(1, D)
    b2r = b2.reshape(1, D)

    s1, ys1 = pl.pallas_call(
        _l1_body,
        out_shape=[jax.ShapeDtypeStruct((NP, D), f32),
                   jax.ShapeDtypeStruct((2 * NP, DH), f32)],
    )(xp, W_self1, W_neigh1, b1r)

    aggp1, degp = _agg_with_deg(ys1, src3d, dst3d, zf, zd)
    degt = degp.T

    s2, ys2, rdeg = pl.pallas_call(
        _l2_body,
        out_shape=[jax.ShapeDtypeStruct((NP, D), f32),
                   jax.ShapeDtypeStruct((2 * NP, DH), f32),
                   jax.ShapeDtypeStruct((NP, 1), f32)],
    )(s1, aggp1, degt, W_self2, W_neigh2, b2r)

    (aggp2,) = _agg_no_deg(ys2, src3d, dst3d, zf)

    out = pl.pallas_call(
        _fin_body,
        out_shape=jax.ShapeDtypeStruct((NP, D), f32),
    )(s2, aggp2, rdeg)
    return out[:N]


# speedup vs baseline: 12.1100x; 1.0481x over previous
import functools

import jax
import jax.numpy as jnp
from jax import lax
from jax.experimental import pallas as pl
from jax.experimental.pallas import tpu as pltpu
from jax.experimental.pallas import tpu_sc as plsc

N = 10000
E = 320000
D = 128
DH = D // 2

NC = 2
NS = 16
C = 125
EPT = E // NS
CPT = EPT // C
NBUF = 5
ROWS_T = N // NS
DEG_T = 1000
DEG_TILES = N // DEG_T
GB = 10
RB = N // GB

_mesh = plsc.VectorSubcoreMesh(core_axis_name="c", subcore_axis_name="s")
_sc_params = pltpu.CompilerParams(use_tc_tiling_on_sc=False)


def _agg_body(with_deg, ys_hbm, edge_hbm, zf_hbm, *rest):
    if with_deg:
        (zd_hbm, aggp_hbm, degp_hbm, src_v, dst_v,
         b0, b1, b2, b3, b4, agg_s, ones_v, deg_s,
         s0, s1, s2, s3, s4) = rest
    else:
        (aggp_hbm, src_v, dst_v,
         b0, b1, b2, b3, b4, agg_s,
         s0, s1, s2, s3, s4) = rest
    bufs = (b0, b1, b2, b3, b4)
    sems = (s0, s1, s2, s3, s4)

    cid = lax.axis_index("c")
    sid = lax.axis_index("s")

    rbase = sid * ROWS_T
    pltpu.sync_copy(zf_hbm, agg_s.at[pl.ds(rbase, ROWS_T)])
    if with_deg:
        @pl.when(sid < DEG_TILES)
        def _():
            pltpu.sync_copy(zd_hbm, deg_s.at[pl.ds(sid * DEG_T, DEG_T)])
        for i in range(8):
            ones_v[pl.ds(i * 16, 16)] = jnp.ones((16,), jnp.float32)

    pltpu.sync_copy(edge_hbm.at[0, sid], src_v)
    pltpu.sync_copy(edge_hbm.at[1, sid], dst_v)

    plsc.subcore_barrier()

    yv = ys_hbm.at[pl.ds(cid * N, N)]

    for b in range(NBUF):
        pltpu.async_copy(yv.at[src_v.at[b]], bufs[b], sems[b])

    def step(c, b):
        pltpu.make_async_copy(yv.at[src_v.at[c]], bufs[b], sems[b]).wait()
        pltpu.sync_copy(bufs[b], agg_s.at[dst_v.at[c]], add=True)
        if with_deg:
            @pl.when((cid == 0) == (c < CPT // 2))
            def _():
                pltpu.sync_copy(ones_v.at[pl.ds(0, C)],
                                deg_s.at[dst_v.at[c]], add=True)

    def outer(i, carry):
        c0 = i * NBUF
        for b in range(NBUF):
            step(c0 + b, b)
            pltpu.async_copy(yv.at[src_v.at[c0 + b + NBUF]], bufs[b], sems[b])
        return carry

    lax.fori_loop(0, CPT // NBUF - 1, outer, 0)
    for b in range(NBUF):
        step(CPT - NBUF + b, b)

    plsc.subcore_barrier()

    pltpu.sync_copy(agg_s.at[pl.ds(rbase, ROWS_T)],
                    aggp_hbm.at[cid, pl.ds(rbase, ROWS_T)])
    if with_deg:
        @pl.when(sid < DEG_TILES)
        def _():
            pltpu.sync_copy(deg_s.at[pl.ds(sid * DEG_T, DEG_T)],
                            degp_hbm.at[cid, pl.ds(sid * DEG_T, DEG_T)])


def _make_agg(with_deg):
    out_type = [jax.ShapeDtypeStruct((NC, N, DH), jnp.float32)]
    if with_deg:
        out_type.append(jax.ShapeDtypeStruct((NC, N), jnp.float32))
    scratch = [
        pltpu.VMEM((CPT, C), jnp.int32),
        pltpu.VMEM((CPT, C), jnp.int32),
    ]
    scratch += [pltpu.VMEM((C, DH), jnp.float32) for _ in range(NBUF)]
    scratch += [pltpu.VMEM_SHARED((N, DH), jnp.float32)]
    if with_deg:
        scratch += [pltpu.VMEM((128,), jnp.float32),
                    pltpu.VMEM_SHARED((N,), jnp.float32)]
    scratch += [pltpu.SemaphoreType.DMA for _ in range(NBUF)]
    return pl.kernel(
        functools.partial(_agg_body, with_deg),
        out_type=out_type,
        mesh=_mesh,
        scratch_types=scratch,
        compiler_params=_sc_params,
    )


_agg_with_deg = _make_agg(True)
_agg_no_deg = _make_agg(False)


def _l1_body(x_ref, ws_ref, wn_ref, b_ref, s_ref, ys_ref):
    xv = x_ref[...]
    s_ref[...] = jnp.dot(xv, ws_ref[...], preferred_element_type=jnp.float32) + b_ref[...]
    y = jnp.dot(xv, wn_ref[...], preferred_element_type=jnp.float32)
    ys_ref[0] = y[:, 0:DH]
    ys_ref[1] = y[:, DH:D]


def _l2_body(s1_ref, aggp_ref, degt_ref, ws_ref, wn_ref, b_ref,
             s2_ref, ys2_ref, rdeg_ref):
    deg = degt_ref[:, 0:1] + degt_ref[:, 1:2]
    rdeg = 1.0 / jnp.maximum(deg, 1.0)
    agg = jnp.concatenate([aggp_ref[0], aggp_ref[1]], axis=1)
    h1 = jnp.maximum(s1_ref[...] + agg * rdeg, 0.0)
    s2_ref[...] = jnp.dot(h1, ws_ref[...], preferred_element_type=jnp.float32) + b_ref[...]
    y2 = jnp.dot(h1, wn_ref[...], preferred_element_type=jnp.float32)
    ys2_ref[0] = y2[:, 0:DH]
    ys2_ref[1] = y2[:, DH:D]
    rdeg_ref[...] = rdeg


def _fin_body(s2_ref, aggp_ref, rdeg_ref, out_ref):
    agg = jnp.concatenate([aggp_ref[0], aggp_ref[1]], axis=1)
    out_ref[...] = s2_ref[...] + agg * rdeg_ref[...]


def kernel(x, edge_index, W_self1, W_neigh1, b1, W_self2, W_neigh2, b2):
    f32 = jnp.float32
    e4 = edge_index.reshape(2, NS, CPT, C)
    zf = jnp.zeros((ROWS_T, DH), f32)
    zd = jnp.zeros((DEG_T,), f32)
    b1r = b1.reshape(1, D)
    b2r = b2.reshape(1, D)

    _row = pl.BlockSpec((RB, D), lambda i: (i, 0))
    _half = pl.BlockSpec((NC, RB, DH), lambda i: (0, i, 0))
    _wspec = pl.BlockSpec((D, D), lambda i: (0, 0))
    _bspec = pl.BlockSpec((1, D), lambda i: (0, 0))
    _col = pl.BlockSpec((RB, 1), lambda i: (i, 0))

    s1, ys1p = pl.pallas_call(
        _l1_body,
        grid=(GB,),
        in_specs=[_row, _wspec, _wspec, _bspec],
        out_specs=[_row, _half],
        out_shape=[jax.ShapeDtypeStruct((N, D), f32),
                   jax.ShapeDtypeStruct((NC, N, DH), f32)],
    )(x, W_self1, W_neigh1, b1r)
    ys1 = ys1p.reshape(2 * N, DH)

    aggp1, degp = _agg_with_deg(ys1, e4, zf, zd)
    degt = degp.T

    s2, ys2p, rdeg = pl.pallas_call(
        _l2_body,
        grid=(GB,),
        in_specs=[_row, _half, pl.BlockSpec((RB, NC), lambda i: (i, 0)),
                  _wspec, _wspec, _bspec],
        out_specs=[_row, _half, _col],
        out_shape=[jax.ShapeDtypeStruct((N, D), f32),
                   jax.ShapeDtypeStruct((NC, N, DH), f32),
                   jax.ShapeDtypeStruct((N, 1), f32)],
    )(s1, aggp1, degt, W_self2, W_neigh2, b2r)
    ys2 = ys2p.reshape(2 * N, DH)

    (aggp2,) = _agg_no_deg(ys2, e4, zf)

    out = pl.pallas_call(
        _fin_body,
        grid=(GB,),
        in_specs=[_row, _half, _col],
        out_specs=_row,
        out_shape=jax.ShapeDtypeStruct((N, D), f32),
    )(s2, aggp2, rdeg)
    return out


# speedup vs baseline: 12.3348x; 1.0186x over previous
import functools

import jax
import jax.numpy as jnp
from jax import lax
from jax.experimental import pallas as pl
from jax.experimental.pallas import tpu as pltpu
from jax.experimental.pallas import tpu_sc as plsc

N = 10000
E = 320000
D = 128
DH = D // 2

NC = 2
NS = 16
NP = 10240
C = 125
EPT = E // NS
CPT = EPT // C
NBUF = 5
ROWS_T = NP // NS

_mesh = plsc.VectorSubcoreMesh(core_axis_name="c", subcore_axis_name="s")
_sc_params = pltpu.CompilerParams(use_tc_tiling_on_sc=False)


def _agg_body(with_deg, ys_hbm, edge_hbm, zf_hbm, *rest):
    if with_deg:
        (zd_hbm, aggp_hbm, degp_hbm, src_v, dst_v,
         b0, b1, b2, b3, b4, agg_s, ones_v, deg_s,
         s0, s1, s2, s3, s4) = rest
    else:
        (aggp_hbm, src_v, dst_v,
         b0, b1, b2, b3, b4, agg_s,
         s0, s1, s2, s3, s4) = rest
    bufs = (b0, b1, b2, b3, b4)
    sems = (s0, s1, s2, s3, s4)

    cid = lax.axis_index("c")
    sid = lax.axis_index("s")

    rbase = sid * ROWS_T
    pltpu.sync_copy(zf_hbm, agg_s.at[pl.ds(rbase, ROWS_T)])
    if with_deg:
        pltpu.sync_copy(zd_hbm, deg_s.at[pl.ds(rbase, ROWS_T)])
        for i in range(8):
            ones_v[pl.ds(i * 16, 16)] = jnp.ones((16,), jnp.float32)

    pltpu.sync_copy(edge_hbm.at[0, sid], src_v)
    pltpu.sync_copy(edge_hbm.at[1, sid], dst_v)

    plsc.subcore_barrier()

    yv = ys_hbm.at[pl.ds(cid * NP, NP)]

    for b in range(NBUF):
        pltpu.async_copy(yv.at[src_v.at[b]], bufs[b], sems[b])

    def step(c, b):
        pltpu.make_async_copy(yv.at[src_v.at[c]], bufs[b], sems[b]).wait()
        pltpu.sync_copy(bufs[b], agg_s.at[dst_v.at[c]], add=True)
        if with_deg:
            @pl.when((cid == 0) == (c < CPT // 2))
            def _():
                pltpu.sync_copy(ones_v.at[pl.ds(0, C)],
                                deg_s.at[dst_v.at[c]], add=True)

    def outer(i, carry):
        c0 = i * NBUF
        for b in range(NBUF):
            step(c0 + b, b)
            pltpu.async_copy(yv.at[src_v.at[c0 + b + NBUF]], bufs[b], sems[b])
        return carry

    lax.fori_loop(0, CPT // NBUF - 1, outer, 0)
    for b in range(NBUF):
        step(CPT - NBUF + b, b)

    plsc.subcore_barrier()

    pltpu.sync_copy(agg_s.at[pl.ds(rbase, ROWS_T)],
                    aggp_hbm.at[cid, pl.ds(rbase, ROWS_T)])
    if with_deg:
        pltpu.sync_copy(deg_s.at[pl.ds(rbase, ROWS_T)],
                        degp_hbm.at[cid, pl.ds(rbase, ROWS_T)])


def _make_agg(with_deg):
    out_type = [jax.ShapeDtypeStruct((NC, NP, DH), jnp.float32)]
    if with_deg:
        out_type.append(jax.ShapeDtypeStruct((NC, NP), jnp.float32))
    scratch = [
        pltpu.VMEM((CPT, C), jnp.int32),
        pltpu.VMEM((CPT, C), jnp.int32),
    ]
    scratch += [pltpu.VMEM((C, DH), jnp.float32) for _ in range(NBUF)]
    scratch += [pltpu.VMEM_SHARED((NP, DH), jnp.float32)]
    if with_deg:
        scratch += [pltpu.VMEM((128,), jnp.float32),
                    pltpu.VMEM_SHARED((NP,), jnp.float32)]
    scratch += [pltpu.SemaphoreType.DMA for _ in range(NBUF)]
    return pl.kernel(
        functools.partial(_agg_body, with_deg),
        out_type=out_type,
        mesh=_mesh,
        scratch_types=scratch,
        compiler_params=_sc_params,
    )


_agg_with_deg = _make_agg(True)
_agg_no_deg = _make_agg(False)


def _l1_body(x_ref, ws_ref, wn_ref, b_ref, s_ref, ys_ref):
    xv = x_ref[...]
    s_ref[0:N, :] = jnp.dot(xv, ws_ref[...], preferred_element_type=jnp.float32) + b_ref[...]
    s_ref[N:NP, :] = jnp.zeros((NP - N, D), jnp.float32)
    y = jnp.dot(xv, wn_ref[...], preferred_element_type=jnp.float32)
    ys_ref[0:N, :] = y[:, 0:DH]
    ys_ref[NP:NP + N, :] = y[:, DH:D]


def _l2_body(s1_ref, aggp_ref, degt_ref, ws_ref, wn_ref, b_ref,
             s2_ref, ys2_ref, rdeg_ref):
    deg = degt_ref[:, 0:1] + degt_ref[:, 1:2]
    rdeg = 1.0 / jnp.maximum(deg, 1.0)
    agg = jnp.concatenate([aggp_ref[0], aggp_ref[1]], axis=1)
    h1 = jnp.maximum(s1_ref[...] + agg * rdeg, 0.0)
    s2_ref[...] = jnp.dot(h1, ws_ref[...], preferred_element_type=jnp.float32) + b_ref[...]
    y2 = jnp.dot(h1, wn_ref[...], preferred_element_type=jnp.float32)
    ys2_ref[0] = y2[:, 0:DH]
    ys2_ref[1] = y2[:, DH:D]
    rdeg_ref[...] = rdeg


def _fin_body(s2_ref, aggp_ref, rdeg_ref, out_ref):
    agg = jnp.concatenate([aggp_ref[0, 0:N], aggp_ref[1, 0:N]], axis=1)
    out_ref[...] = s2_ref[0:N] + agg * rdeg_ref[0:N]


def kernel(x, edge_index, W_self1, W_neigh1, b1, W_self2, W_neigh2, b2):
    f32 = jnp.float32
    e4 = edge_index.reshape(2, NS, CPT, C)
    zf = jnp.zeros((ROWS_T, DH), f32)
    zd = jnp.zeros((ROWS_T,), f32)
    b1r = b1.reshape(1, D)
    b2r = b2.reshape(1, D)

    s1, ys1 = pl.pallas_call(
        _l1_body,
        out_shape=[jax.ShapeDtypeStruct((NP, D), f32),
                   jax.ShapeDtypeStruct((2 * NP, DH), f32)],
    )(x, W_self1, W_neigh1, b1r)

    aggp1, degp = _agg_with_deg(ys1, e4, zf, zd)
    degt = degp.T

    RB = NP // 8
    s2, ys2p, rdeg = pl.pallas_call(
        _l2_body,
        grid=(8,),
        in_specs=[pl.BlockSpec((RB, D), lambda i: (i, 0)),
                  pl.BlockSpec((NC, RB, DH), lambda i: (0, i, 0)),
                  pl.BlockSpec((RB, NC), lambda i: (i, 0)),
                  pl.BlockSpec((D, D), lambda i: (0, 0)),
                  pl.BlockSpec((D, D), lambda i: (0, 0)),
                  pl.BlockSpec((1, D), lambda i: (0, 0))],
        out_specs=[pl.BlockSpec((RB, D), lambda i: (i, 0)),
                   pl.BlockSpec((NC, RB, DH), lambda i: (0, i, 0)),
                   pl.BlockSpec((RB, 1), lambda i: (i, 0))],
        out_shape=[jax.ShapeDtypeStruct((NP, D), f32),
                   jax.ShapeDtypeStruct((NC, NP, DH), f32),
                   jax.ShapeDtypeStruct((NP, 1), f32)],
    )(s1, aggp1, degt, W_self2, W_neigh2, b2r)
    ys2 = ys2p.reshape(2 * NP, DH)

    (aggp2,) = _agg_no_deg(ys2, e4, zf)

    out = pl.pallas_call(
        _fin_body,
        out_shape=jax.ShapeDtypeStruct((N, D), f32),
    )(s2, aggp2, rdeg)
    return out


# speedup vs baseline: 12.3441x; 1.0008x over previous
import functools

import jax
import jax.numpy as jnp
from jax import lax
from jax.experimental import pallas as pl
from jax.experimental.pallas import tpu as pltpu
from jax.experimental.pallas import tpu_sc as plsc

N = 10000
E = 320000
D = 128
DH = D // 2

NC = 2
NS = 16
NP = 10240
C = 125
EPT = E // NS
CPT = EPT // C
NBUF = 5
ROWS_T = NP // NS

_mesh = plsc.VectorSubcoreMesh(core_axis_name="c", subcore_axis_name="s")
_sc_params = pltpu.CompilerParams(use_tc_tiling_on_sc=False)


def _agg_body(with_deg, ys_hbm, edge_hbm, zf_hbm, *rest):
    rest = list(rest)
    if with_deg:
        zd_hbm, aggp_hbm, degp_hbm = rest[:3]
        src_v, dst_v = rest[3:5]
        bufs = rest[5:5 + NBUF]
        agg_s, ones_v, deg_s = rest[5 + NBUF:8 + NBUF]
        sems = rest[8 + NBUF:]
    else:
        aggp_hbm, src_v, dst_v = rest[:3]
        bufs = rest[3:3 + NBUF]
        agg_s = rest[3 + NBUF]
        sems = rest[4 + NBUF:]

    cid = lax.axis_index("c")
    sid = lax.axis_index("s")

    rbase = sid * ROWS_T
    pltpu.sync_copy(zf_hbm, agg_s.at[pl.ds(rbase, ROWS_T)])
    if with_deg:
        pltpu.sync_copy(zd_hbm, deg_s.at[pl.ds(rbase, ROWS_T)])
        for i in range(8):
            ones_v[pl.ds(i * 16, 16)] = jnp.ones((16,), jnp.float32)

    pltpu.sync_copy(edge_hbm.at[0, sid], src_v)
    pltpu.sync_copy(edge_hbm.at[1, sid], dst_v)

    plsc.subcore_barrier()

    yv = ys_hbm.at[pl.ds(cid * NP, NP)]

    for b in range(NBUF):
        pltpu.async_copy(yv.at[src_v.at[b]], bufs[b], sems[b])

    def step(c, b):
        pltpu.make_async_copy(yv.at[src_v.at[c]], bufs[b], sems[b]).wait()
        pltpu.sync_copy(bufs[b], agg_s.at[dst_v.at[c]], add=True)
        if with_deg:
            @pl.when((cid == 0) == (c < CPT // 2))
            def _():
                pltpu.sync_copy(ones_v.at[pl.ds(0, C)],
                                deg_s.at[dst_v.at[c]], add=True)

    def outer(i, carry):
        c0 = i * NBUF
        for b in range(NBUF):
            step(c0 + b, b)
            pltpu.async_copy(yv.at[src_v.at[c0 + b + NBUF]], bufs[b], sems[b])
        return carry

    lax.fori_loop(0, CPT // NBUF - 1, outer, 0)
    for b in range(NBUF):
        step(CPT - NBUF + b, b)

    plsc.subcore_barrier()

    pltpu.sync_copy(agg_s.at[pl.ds(rbase, ROWS_T)],
                    aggp_hbm.at[cid, pl.ds(rbase, ROWS_T)])
    if with_deg:
        pltpu.sync_copy(deg_s.at[pl.ds(rbase, ROWS_T)],
                        degp_hbm.at[cid, pl.ds(rbase, ROWS_T)])


def _make_agg(with_deg):
    out_type = [jax.ShapeDtypeStruct((NC, NP, DH), jnp.float32)]
    if with_deg:
        out_type.append(jax.ShapeDtypeStruct((NC, NP), jnp.float32))
    scratch = [
        pltpu.VMEM((CPT, C), jnp.int32),
        pltpu.VMEM((CPT, C), jnp.int32),
    ]
    scratch += [pltpu.VMEM((C, DH), jnp.float32) for _ in range(NBUF)]
    scratch += [pltpu.VMEM_SHARED((NP, DH), jnp.float32)]
    if with_deg:
        scratch += [pltpu.VMEM((128,), jnp.float32),
                    pltpu.VMEM_SHARED((NP,), jnp.float32)]
    scratch += [pltpu.SemaphoreType.DMA for _ in range(NBUF)]
    return pl.kernel(
        functools.partial(_agg_body, with_deg),
        out_type=out_type,
        mesh=_mesh,
        scratch_types=scratch,
        compiler_params=_sc_params,
    )


_agg_with_deg = _make_agg(True)
_agg_no_deg = _make_agg(False)


def _l1_body(x_ref, ws_ref, wn_ref, b_ref, s_ref, ys_ref):
    xv = x_ref[...]
    s_ref[0:N, :] = jnp.dot(xv, ws_ref[...], preferred_element_type=jnp.float32) + b_ref[...]
    s_ref[N:NP, :] = jnp.zeros((NP - N, D), jnp.float32)
    y = jnp.dot(xv, wn_ref[...], preferred_element_type=jnp.float32)
    ys_ref[0:N, :] = y[:, 0:DH]
    ys_ref[NP:NP + N, :] = y[:, DH:D]


def _l2_body(s1_ref, aggp_ref, degt_ref, ws_ref, wn_ref, b_ref,
             s2_ref, ys2_ref, rdeg_ref):
    deg = degt_ref[:, 0:1] + degt_ref[:, 1:2]
    rdeg = 1.0 / jnp.maximum(deg, 1.0)
    agg = jnp.concatenate([aggp_ref[0], aggp_ref[1]], axis=1)
    h1 = jnp.maximum(s1_ref[...] + agg * rdeg, 0.0)
    s2_ref[...] = jnp.dot(h1, ws_ref[...], preferred_element_type=jnp.float32) + b_ref[...]
    y2 = jnp.dot(h1, wn_ref[...], preferred_element_type=jnp.float32)
    ys2_ref[0] = y2[:, 0:DH]
    ys2_ref[1] = y2[:, DH:D]
    rdeg_ref[...] = rdeg


def _fin_body(s2_ref, aggp_ref, rdeg_ref, out_ref):
    agg = jnp.concatenate([aggp_ref[0, 0:N], aggp_ref[1, 0:N]], axis=1)
    out_ref[...] = s2_ref[0:N] + agg * rdeg_ref[0:N]


def kernel(x, edge_index, W_self1, W_neigh1, b1, W_self2, W_neigh2, b2):
    f32 = jnp.float32
    e4 = edge_index.reshape(2, NS, CPT, C)
    zf = jnp.zeros((ROWS_T, DH), f32)
    zd = jnp.zeros((ROWS_T,), f32)
    b1r = b1.reshape(1, D)
    b2r = b2.reshape(1, D)

    s1, ys1 = pl.pallas_call(
        _l1_body,
        out_shape=[jax.ShapeDtypeStruct((NP, D), f32),
                   jax.ShapeDtypeStruct((2 * NP, DH), f32)],
    )(x, W_self1, W_neigh1, b1r)

    aggp1, degp = _agg_with_deg(ys1, e4, zf, zd)
    degt = degp.T

    RB = NP // 8
    s2, ys2p, rdeg = pl.pallas_call(
        _l2_body,
        grid=(8,),
        in_specs=[pl.BlockSpec((RB, D), lambda i: (i, 0)),
                  pl.BlockSpec((NC, RB, DH), lambda i: (0, i, 0)),
                  pl.BlockSpec((RB, NC), lambda i: (i, 0)),
                  pl.BlockSpec((D, D), lambda i: (0, 0)),
                  pl.BlockSpec((D, D), lambda i: (0, 0)),
                  pl.BlockSpec((1, D), lambda i: (0, 0))],
        out_specs=[pl.BlockSpec((RB, D), lambda i: (i, 0)),
                   pl.BlockSpec((NC, RB, DH), lambda i: (0, i, 0)),
                   pl.BlockSpec((RB, 1), lambda i: (i, 0))],
        out_shape=[jax.ShapeDtypeStruct((NP, D), f32),
                   jax.ShapeDtypeStruct((NC, NP, DH), f32),
                   jax.ShapeDtypeStruct((NP, 1), f32)],
    )(s1, aggp1, degt, W_self2, W_neigh2, b2r)
    ys2 = ys2p.reshape(2 * NP, DH)

    (aggp2,) = _agg_no_deg(ys2, e4, zf)

    out = pl.pallas_call(
        _fin_body,
        out_shape=jax.ShapeDtypeStruct((N, D), f32),
    )(s2, aggp2, rdeg)
    return out
